# Initial kernel scaffold; baseline (speedup 1.0000x reference)
import functools

import jax
import jax.numpy as jnp
from jax import lax
from jax.experimental import pallas as pl
from jax.experimental.pallas import tpu as pltpu
from jax.experimental.pallas import tpu_sc as plsc

N = 10000
E = 320000
H = 128
G = 64
D_OUT = 16
EPS = 1e-5

NC = 2
NS = 16
NW = NC * NS
C = 128
NCHUNK = E // C
ITERS = -(-NCHUNK // NW)
ROWS_PER_TILE = N // NS
ZR = 125


def _spmm_body(hw_ref, src_ref, dst_ref, out_ref, idxs, idxd, rows, zbuf, acc,
               sem):
    cid = lax.axis_index("c")
    sid = lax.axis_index("s")

    def _zrow(i, carry):
        for j in range(H // 16):
            zbuf[i, pl.ds(j * 16, 16)] = jnp.zeros((16,), jnp.float32)
        return carry

    lax.fori_loop(0, ZR, _zrow, 0)
    row0 = sid * ROWS_PER_TILE
    for k in range(ROWS_PER_TILE // ZR):
        pltpu.sync_copy(zbuf, acc.at[pl.ds(row0 + k * ZR, ZR)])
    plsc.subcore_barrier()

    wid = sid * NC + cid

    def _edge(t, carry):
        ch = wid + t * NW

        @pl.when(ch < NCHUNK)
        def _():
            base = pl.multiple_of(ch * C, 8)
            pltpu.sync_copy(src_ref.at[pl.ds(base, C)], idxs)
            pltpu.sync_copy(dst_ref.at[pl.ds(base, C)], idxd)
            pltpu.async_copy(hw_ref.at[idxs], rows, sem).wait()
            pltpu.sync_copy(rows, acc.at[idxd], add=True)

        return carry

    lax.fori_loop(0, ITERS, _edge, 0)
    plsc.subcore_barrier()

    for k in range(ROWS_PER_TILE // ZR):
        r = row0 + k * ZR
        pltpu.sync_copy(acc.at[pl.ds(r, ZR)], zbuf)
        pltpu.sync_copy(zbuf, out_ref.at[cid, pl.ds(r, ZR), :])


def _spmm_sc(hw, src, dst):
    mesh = plsc.VectorSubcoreMesh(core_axis_name="c", subcore_axis_name="s")
    kern = pl.kernel(
        _spmm_body,
        out_type=jax.ShapeDtypeStruct((NC, N, H), jnp.float32),
        mesh=mesh,
        scratch_types=[
            pltpu.VMEM((C,), jnp.int32),
            pltpu.VMEM((C,), jnp.int32),
            pltpu.VMEM((C, H), jnp.float32),
            pltpu.VMEM((ZR, H), jnp.float32),
            pltpu.VMEM_SHARED((N, H), jnp.float32),
            pltpu.SemaphoreType.DMA,
        ],
    )
    return kern(hw, src, dst)


def _matmul_t(a, w):
    return lax.dot_general(a, w, (((1,), (1,)), ((), ())),
                           preferred_element_type=jnp.float32)


def _tc_in_body(x_ref, wi_ref, bi_ref, w0_ref, b0_ref, h_ref, hw_ref):
    h = jnp.maximum(_matmul_t(x_ref[...], wi_ref[...]) + bi_ref[...], 0.0)
    h_ref[...] = h
    hw_ref[...] = _matmul_t(h, w0_ref[...]) + b0_ref[...]


def _tc_in(x, W_in, b_in, W0, b0):
    return pl.pallas_call(
        _tc_in_body,
        out_shape=[
            jax.ShapeDtypeStruct((N, H), jnp.float32),
            jax.ShapeDtypeStruct((N, H), jnp.float32),
        ],
    )(x, W_in, b_in, W0, b0)


def _bn_relu_res(h, p0, p1, g, be):
    agg = p0 + p1
    mean = jnp.mean(agg, axis=0, keepdims=True)
    cen = agg - mean
    var = jnp.mean(cen * cen, axis=0, keepdims=True)
    hn = cen * lax.rsqrt(var + EPS) * g + be
    return h + jnp.maximum(hn, 0.0)


def _tc_mid_body(h_ref, p0_ref, p1_ref, g_ref, be_ref, wn_ref, bn_ref,
                 h_out_ref, hw_ref):
    h_new = _bn_relu_res(h_ref[...], p0_ref[...], p1_ref[...], g_ref[...],
                         be_ref[...])
    h_out_ref[...] = h_new
    hw_ref[...] = _matmul_t(h_new, wn_ref[...]) + bn_ref[...]


def _tc_mid(h, p0, p1, g, be, Wn, bn_):
    return pl.pallas_call(
        _tc_mid_body,
        out_shape=[
            jax.ShapeDtypeStruct((N, H), jnp.float32),
            jax.ShapeDtypeStruct((N, H), jnp.float32),
        ],
    )(h, p0, p1, g, be, Wn, bn_)


def _tc_out_body(h_ref, p0_ref, p1_ref, g_ref, be_ref, bcol_ref, wg_ref,
                 bg_ref, w1_ref, b1_ref, w2_ref, b2_ref, out_ref):
    h_f = _bn_relu_res(h_ref[...], p0_ref[...], p1_ref[...], g_ref[...],
                       be_ref[...])
    gate = jnp.sum(h_f * wg_ref[...], axis=1, keepdims=True) + bg_ref[...]
    bcol = bcol_ref[...]
    seg = lax.broadcasted_iota(jnp.int32, (N, G), 1)
    mask = bcol == seg
    neg = jnp.float32(-1e30)
    gm = jnp.max(jnp.where(mask, gate, neg), axis=0, keepdims=True)
    gmb = jnp.max(jnp.where(mask, gm, neg), axis=1, keepdims=True)
    expv = jnp.exp(gate - gmb)
    oh = mask.astype(jnp.float32)
    ssum = jnp.sum(oh * expv, axis=0, keepdims=True)
    ssb = jnp.sum(oh * ssum, axis=1, keepdims=True)
    alpha = expv / (ssb + 1e-10)
    wt = h_f * alpha
    ge = lax.dot_general(oh, wt, (((0,), (0,)), ((), ())),
                         preferred_element_type=jnp.float32)
    e1 = jnp.maximum(_matmul_t(ge, w1_ref[...]) + b1_ref[...], 0.0)
    out_ref[...] = _matmul_t(e1, w2_ref[...]) + b2_ref[...]


def _tc_out(h, p0, p1, g, be, bcol, Wg, bg, W1, b1, W2, b2):
    return pl.pallas_call(
        _tc_out_body,
        out_shape=jax.ShapeDtypeStruct((G, D_OUT), jnp.float32),
    )(h, p0, p1, g, be, bcol, Wg, bg, W1, b1, W2, b2)


def kernel(x, edge_index, batch, W_in, b_in,
           W_conv0, b_conv0, bn_g0, bn_b0,
           W_conv1, b_conv1, bn_g1, bn_b1,
           W_conv2, b_conv2, bn_g2, bn_b2,
           W_gate, b_gate, W_h1, b_h1, W_h2, b_h2):
    src = edge_index[0]
    dst = edge_index[1]
    bcol = batch.reshape(N, 1)

    h, hw = _tc_in(x, W_in, b_in.reshape(1, H), W_conv0,
                   b_conv0.reshape(1, H))
    layers = ((bn_g0, bn_b0, W_conv1, b_conv1),
              (bn_g1, bn_b1, W_conv2, b_conv2))
    for (g, be, Wn, bn_) in layers:
        p = _spmm_sc(hw, src, dst)
        h, hw = _tc_mid(h, p[0], p[1], g.reshape(1, H), be.reshape(1, H),
                        Wn, bn_.reshape(1, H))
    p = _spmm_sc(hw, src, dst)
    out = _tc_out(h, p[0], p[1], bn_g2.reshape(1, H), bn_b2.reshape(1, H),
                  bcol, W_gate, b_gate.reshape(1, 1),
                  W_h1, b_h1.reshape(1, H), W_h2, b_h2.reshape(1, D_OUT))
    return out


# speedup vs baseline: 5.7757x; 5.7757x over previous
import functools

import jax
import jax.numpy as jnp
from jax import lax
from jax.experimental import pallas as pl
from jax.experimental.pallas import tpu as pltpu
from jax.experimental.pallas import tpu_sc as plsc

N = 10000
E = 320000
H = 128
G = 64
D_OUT = 16
EPS = 1e-5

NC = 2
NS = 16
NW = NC * NS
C = 128
NCHUNK = E // C
ITERS = -(-NCHUNK // NW)
NP = 10240
ROWS_PER_TILE = NP // NS
ZR = 128


def _spmm_body(hw_ref, src_ref, dst_ref, out_ref, idxs, idxd, rows, zbuf, acc,
               sem):
    cid = lax.axis_index("c")
    sid = lax.axis_index("s")

    def _zrow(i, carry):
        for j in range(H // 16):
            zbuf[i, pl.ds(j * 16, 16)] = jnp.zeros((16,), jnp.float32)
        return carry

    lax.fori_loop(0, ZR, _zrow, 0)
    row0 = sid * ROWS_PER_TILE
    for k in range(ROWS_PER_TILE // ZR):
        pltpu.sync_copy(zbuf, acc.at[pl.ds(row0 + k * ZR, ZR)])
    plsc.subcore_barrier()

    wid = sid * NC + cid

    def _edge(t, carry):
        ch = wid + t * NW

        @pl.when(ch < NCHUNK)
        def _():
            base = pl.multiple_of(ch * C, 8)
            pltpu.sync_copy(src_ref.at[pl.ds(base, C)], idxs)
            pltpu.sync_copy(dst_ref.at[pl.ds(base, C)], idxd)
            pltpu.async_copy(hw_ref.at[idxs], rows, sem).wait()
            pltpu.sync_copy(rows, acc.at[idxd], add=True)

        return carry

    lax.fori_loop(0, ITERS, _edge, 0)
    plsc.subcore_barrier()

    for k in range(ROWS_PER_TILE // ZR):
        r = row0 + k * ZR
        pltpu.sync_copy(acc.at[pl.ds(r, ZR)], zbuf)
        pltpu.sync_copy(zbuf, out_ref.at[cid, pl.ds(r, ZR), :])


def _spmm_sc(hw, src, dst):
    mesh = plsc.VectorSubcoreMesh(core_axis_name="c", subcore_axis_name="s")
    kern = pl.kernel(
        _spmm_body,
        out_type=jax.ShapeDtypeStruct((NC, NP, H), jnp.float32),
        mesh=mesh,
        scratch_types=[
            pltpu.VMEM((C,), jnp.int32),
            pltpu.VMEM((C,), jnp.int32),
            pltpu.VMEM((C, H), jnp.float32),
            pltpu.VMEM((ZR, H), jnp.float32),
            pltpu.VMEM_SHARED((NP, H), jnp.float32),
            pltpu.SemaphoreType.DMA,
        ],
    )
    return kern(hw, src, dst)


def _matmul_t(a, w):
    return lax.dot_general(a, w, (((1,), (1,)), ((), ())),
                           preferred_element_type=jnp.float32)


def _tc_in_body(x_ref, wi_ref, bi_ref, w0_ref, b0_ref, h_ref, hw_ref):
    h = jnp.maximum(_matmul_t(x_ref[...], wi_ref[...]) + bi_ref[...], 0.0)
    h_ref[...] = h
    hw_ref[...] = _matmul_t(h, w0_ref[...]) + b0_ref[...]


def _tc_in(x, W_in, b_in, W0, b0):
    return pl.pallas_call(
        _tc_in_body,
        out_shape=[
            jax.ShapeDtypeStruct((N, H), jnp.float32),
            jax.ShapeDtypeStruct((N, H), jnp.float32),
        ],
    )(x, W_in, b_in, W0, b0)


def _bn_relu_res(h, p0, p1, g, be):
    agg = p0[:N] + p1[:N]
    mean = jnp.mean(agg, axis=0, keepdims=True)
    cen = agg - mean
    var = jnp.mean(cen * cen, axis=0, keepdims=True)
    hn = cen * lax.rsqrt(var + EPS) * g + be
    return h + jnp.maximum(hn, 0.0)


def _tc_mid_body(h_ref, p0_ref, p1_ref, g_ref, be_ref, wn_ref, bn_ref,
                 h_out_ref, hw_ref):
    h_new = _bn_relu_res(h_ref[...], p0_ref[...], p1_ref[...], g_ref[...],
                         be_ref[...])
    h_out_ref[...] = h_new
    hw_ref[...] = _matmul_t(h_new, wn_ref[...]) + bn_ref[...]


def _tc_mid(h, p0, p1, g, be, Wn, bn_):
    return pl.pallas_call(
        _tc_mid_body,
        out_shape=[
            jax.ShapeDtypeStruct((N, H), jnp.float32),
            jax.ShapeDtypeStruct((N, H), jnp.float32),
        ],
    )(h, p0, p1, g, be, Wn, bn_)


def _tc_out_body(h_ref, p0_ref, p1_ref, g_ref, be_ref, bcol_ref, wg_ref,
                 bg_ref, w1_ref, b1_ref, w2_ref, b2_ref, out_ref):
    h_f = _bn_relu_res(h_ref[...], p0_ref[...], p1_ref[...], g_ref[...],
                       be_ref[...])
    gate = jnp.sum(h_f * wg_ref[...], axis=1, keepdims=True) + bg_ref[...]
    bcol = bcol_ref[...]
    seg = lax.broadcasted_iota(jnp.int32, (N, G), 1)
    mask = bcol == seg
    neg = jnp.float32(-1e30)
    gm = jnp.max(jnp.where(mask, gate, neg), axis=0, keepdims=True)
    gmb = jnp.max(jnp.where(mask, gm, neg), axis=1, keepdims=True)
    expv = jnp.exp(gate - gmb)
    oh = mask.astype(jnp.float32)
    ssum = jnp.sum(oh * expv, axis=0, keepdims=True)
    ssb = jnp.sum(oh * ssum, axis=1, keepdims=True)
    alpha = expv / (ssb + 1e-10)
    wt = h_f * alpha
    ge = lax.dot_general(oh, wt, (((0,), (0,)), ((), ())),
                         preferred_element_type=jnp.float32)
    e1 = jnp.maximum(_matmul_t(ge, w1_ref[...]) + b1_ref[...], 0.0)
    out_ref[...] = _matmul_t(e1, w2_ref[...]) + b2_ref[...]


def _tc_out(h, p0, p1, g, be, bcol, Wg, bg, W1, b1, W2, b2):
    return pl.pallas_call(
        _tc_out_body,
        out_shape=jax.ShapeDtypeStruct((G, D_OUT), jnp.float32),
    )(h, p0, p1, g, be, bcol, Wg, bg, W1, b1, W2, b2)


def kernel(x, edge_index, batch, W_in, b_in,
           W_conv0, b_conv0, bn_g0, bn_b0,
           W_conv1, b_conv1, bn_g1, bn_b1,
           W_conv2, b_conv2, bn_g2, bn_b2,
           W_gate, b_gate, W_h1, b_h1, W_h2, b_h2):
    src = edge_index[0]
    dst = edge_index[1]
    bcol = batch.reshape(N, 1)

    h, hw = _tc_in(x, W_in, b_in.reshape(1, H), W_conv0,
                   b_conv0.reshape(1, H))
    layers = ((bn_g0, bn_b0, W_conv1, b_conv1),
              (bn_g1, bn_b1, W_conv2, b_conv2))
    for (g, be, Wn, bn_) in layers:
        p = _spmm_sc(hw, src, dst)
        h, hw = _tc_mid(h, p[0], p[1], g.reshape(1, H), be.reshape(1, H),
                        Wn, bn_.reshape(1, H))
    p = _spmm_sc(hw, src, dst)
    out = _tc_out(h, p[0], p[1], bn_g2.reshape(1, H), bn_b2.reshape(1, H),
                  bcol, W_gate, b_gate.reshape(1, 1),
                  W_h1, b_h1.reshape(1, H), W_h2, b_h2.reshape(1, D_OUT))
    return out
